# baseline (speedup 1.0000x reference)
import functools

import jax
import jax.numpy as jnp
from jax import lax
from jax.experimental import pallas as pl
from jax.experimental.pallas import tpu as pltpu
from jax.experimental.pallas import tpu_sc as plsc

FIELDS = 26
EMB = 16
NC = 2
NS = 16
NW = NC * NS
L = 16
CHUNK = 128


@functools.partial(jax.jit, static_argnames=("B",))
def _deepfm_sc(fm2flat, fm1flat, idx_arr, xv_arr, w_eff, *, B):
    b_per_w = B // NW
    FV = b_per_w * FIELDS
    n_chunks = FV // CHUNK

    mesh = plsc.VectorSubcoreMesh(
        core_axis_name="c", subcore_axis_name="s",
        num_cores=NC, num_subcores=NS)

    @functools.partial(
        pl.kernel,
        out_type=jax.ShapeDtypeStruct((B,), jnp.float32),
        mesh=mesh,
        scratch_types=[
            pltpu.VMEM((n_chunks, CHUNK), jnp.int32),
            pltpu.VMEM((FV,), jnp.float32),
            pltpu.VMEM((FV, EMB), jnp.float32),
            pltpu.VMEM((FV + L,), jnp.float32),
            pltpu.VMEM((FIELDS * EMB,), jnp.float32),
            pltpu.VMEM((b_per_w,), jnp.float32),
            pltpu.SemaphoreType.DMA,
        ],
        compiler_params=pltpu.CompilerParams(
            use_tc_tiling_on_sc=False, needs_layout_passes=False),
    )
    def k(fm2_hbm, fm1_hbm, idx_hbm, xv_hbm, w_hbm, out_hbm,
          idx_v, xv_v, rows_v, fm1_v, w_v, out_v, sem):
        wid = lax.axis_index("s") * NC + lax.axis_index("c")

        pltpu.sync_copy(idx_hbm.at[wid], idx_v)
        pltpu.sync_copy(xv_hbm.at[wid], xv_v)
        pltpu.sync_copy(w_hbm, w_v)

        copies = []
        for j in range(n_chunks):
            copies.append(pltpu.async_copy(
                fm2_hbm.at[idx_v.at[j]],
                rows_v.at[pl.ds(j * CHUNK, CHUNK)], sem))
            copies.append(pltpu.async_copy(
                fm1_hbm.at[idx_v.at[j]],
                fm1_v.at[pl.ds(j * CHUNK, CHUNK)], sem))
        for cp in copies:
            cp.wait()

        lanes = lax.iota(jnp.int32, L)
        tail_mask = jnp.where(lanes < (FIELDS - L),
                              jnp.float32(1), jnp.float32(0))
        zero = jnp.zeros((L,), jnp.float32)

        fm1_v[pl.ds(FV, L)] = zero

        def prod_body(i, _):
            s = i * L
            fm1_v[pl.ds(s, L)] = fm1_v[pl.ds(s, L)] * xv_v[pl.ds(s, L)]
            return 0
        lax.fori_loop(0, FV // L, prod_body, 0)

        def sample_body(s, ovec):
            base = s * FIELDS
            acc = zero
            sq = zero
            dp = zero
            xa = xv_v[pl.ds(base, L)]
            xb = xv_v[pl.ds(base + L, L)]
            for f in range(FIELDS):
                row = rows_v[base + f, :]
                xf = xa[f] if f < L else xb[f - L]
                v = jnp.full((L,), xf, jnp.float32)
                sr = row * v
                acc = acc + sr
                sq = sq + sr * sr
                dp = dp + sr * w_v[pl.ds(f * EMB, EMB)]
            g1 = fm1_v[pl.ds(base, L)]
            g2 = fm1_v[pl.ds(base + L, L)]
            res = dp + 0.5 * (acc * acc - sq) + g1 + g2 * tail_mask
            total = jnp.sum(res)
            ovec = jnp.where(lanes == (s % L), jnp.full((L,), total), ovec)

            @pl.when(s % L == L - 1)
            def _flush():
                out_v[pl.ds(s - (L - 1), L)] = ovec
            return ovec
        lax.fori_loop(0, b_per_w, sample_body, zero)

        pltpu.sync_copy(out_v, out_hbm.at[pl.ds(wid * b_per_w, b_per_w)])

    return k(fm2flat, fm1flat, idx_arr, xv_arr, w_eff)


def kernel(Xi, Xv, fm1, fm2, W1, b1, g1, be1, W2, b2, g2, be2, bias):
    B = Xv.shape[0]
    vocab = fm2.shape[1]
    b_per_w = B // NW
    FV = b_per_w * FIELDS

    idx = Xi[:, :, 0]
    offs = (jnp.arange(FIELDS, dtype=jnp.int32) * vocab)[None, :]
    idx_arr = (idx + offs).reshape(NW, FV // CHUNK, CHUNK)
    xv_arr = Xv.reshape(NW, FV)
    fm2flat = fm2.reshape(FIELDS * vocab, EMB)
    fm1flat = fm1.reshape(FIELDS * vocab)

    c = 1.0 / jnp.sqrt(jnp.float32(1.0 + 1e-5))
    u = W2.T @ g2
    gu = g1 * u
    w_eff = (c * c) * (W1.T @ gu)
    const = ((c * c) * jnp.dot(b1, gu) + c * jnp.dot(be1, u)
             + c * jnp.dot(b2, g2) + jnp.sum(be2) + bias[0])

    out = _deepfm_sc(fm2flat, fm1flat, idx_arr, xv_arr, w_eff, B=B)
    return out + const


# speedup vs baseline: 1.0020x; 1.0020x over previous
import functools

import jax
import jax.numpy as jnp
from jax import lax
from jax.experimental import pallas as pl
from jax.experimental.pallas import tpu as pltpu
from jax.experimental.pallas import tpu_sc as plsc

FIELDS = 26
EMB = 16
NC = 2
NS = 16
NW = NC * NS
L = 16


@functools.partial(jax.jit, static_argnames=("B",))
def _deepfm_sc(fm2, fm1r, idx_fm, xv_sm, xv_fm, w_eff, *, B):
    b_per_w = B // NW
    FV = b_per_w * FIELDS
    n_groups = b_per_w // L

    mesh = plsc.VectorSubcoreMesh(
        core_axis_name="c", subcore_axis_name="s",
        num_cores=NC, num_subcores=NS)

    @functools.partial(
        pl.kernel,
        out_type=jax.ShapeDtypeStruct((B,), jnp.float32),
        mesh=mesh,
        scratch_types=[
            pltpu.VMEM((FIELDS, b_per_w), jnp.int32),
            pltpu.VMEM((FV + L,), jnp.float32),
            pltpu.VMEM((FV,), jnp.float32),
            pltpu.VMEM((FV, EMB), jnp.float32),
            pltpu.VMEM((FV,), jnp.float32),
            pltpu.VMEM((FIELDS * EMB,), jnp.float32),
            pltpu.VMEM((b_per_w,), jnp.float32),
            pltpu.VMEM((b_per_w,), jnp.float32),
            pltpu.SemaphoreType.DMA,
        ],
        compiler_params=pltpu.CompilerParams(
            use_tc_tiling_on_sc=False, needs_layout_passes=False),
    )
    def k(fm2_hbm, fm1_hbm, idx_hbm, xvs_hbm, xvf_hbm, w_hbm, out_hbm,
          idx_v, xvs_v, xvf_v, rows_v, f1_v, w_v, s1_v, out_v, sem):
        wid = lax.axis_index("s") * NC + lax.axis_index("c")

        pltpu.sync_copy(idx_hbm.at[wid], idx_v)
        pltpu.sync_copy(xvs_hbm.at[wid], xvs_v.at[pl.ds(0, FV)])
        pltpu.sync_copy(xvf_hbm.at[wid], xvf_v)
        pltpu.sync_copy(w_hbm, w_v)

        copies = []
        for f in range(FIELDS):
            copies.append(pltpu.async_copy(
                fm2_hbm.at[f].at[idx_v.at[f]],
                rows_v.at[pl.ds(f * b_per_w, b_per_w)], sem))
            copies.append(pltpu.async_copy(
                fm1_hbm.at[f].at[idx_v.at[f]],
                f1_v.at[pl.ds(f * b_per_w, b_per_w)], sem))
        for cp in copies:
            cp.wait()

        lanes = lax.iota(jnp.int32, L)
        zero = jnp.zeros((L,), jnp.float32)

        def s1_body(g, _):
            off = g * L
            s1 = zero
            for f in range(FIELDS):
                p = pl.ds(f * b_per_w + off, L)
                s1 = s1 + f1_v[p] * xvf_v[p]
            s1_v[pl.ds(off, L)] = s1
            return 0
        lax.fori_loop(0, n_groups, s1_body, 0)

        def sample_body(s, ovec):
            base = s * FIELDS
            acc = zero
            sq = zero
            dp = zero
            xa = xvs_v[pl.ds(base, L)]
            xb = xvs_v[pl.ds(base + L, L)]
            for f in range(FIELDS):
                row = rows_v[f * b_per_w + s, :]
                xf = xa[f] if f < L else xb[f - L]
                v = jnp.full((L,), xf, jnp.float32)
                sr = row * v
                acc = acc + sr
                sq = sq + sr * sr
                dp = dp + sr * w_v[pl.ds(f * EMB, EMB)]
            res = dp + 0.5 * (acc * acc - sq)
            total = jnp.sum(res)
            ovec = jnp.where(lanes == (s % L), jnp.full((L,), total), ovec)

            @pl.when(s % L == L - 1)
            def _flush():
                g0 = s - (L - 1)
                out_v[pl.ds(g0, L)] = ovec + s1_v[pl.ds(g0, L)]
            return ovec
        lax.fori_loop(0, b_per_w, sample_body, zero)

        pltpu.sync_copy(out_v, out_hbm.at[pl.ds(wid * b_per_w, b_per_w)])

    return k(fm2, fm1r, idx_fm, xv_sm, xv_fm, w_eff)


def kernel(Xi, Xv, fm1, fm2, W1, b1, g1, be1, W2, b2, g2, be2, bias):
    B = Xv.shape[0]
    vocab = fm2.shape[1]
    b_per_w = B // NW
    FV = b_per_w * FIELDS

    idxs = Xi[:, :, 0]
    idx_fm = idxs.reshape(NW, b_per_w, FIELDS).transpose(0, 2, 1)
    xv_sm = Xv.reshape(NW, FV)
    xv_fm = Xv.reshape(NW, b_per_w, FIELDS).transpose(0, 2, 1).reshape(NW, FV)
    fm1r = fm1[:, :, 0]

    c = 1.0 / jnp.sqrt(jnp.float32(1.0 + 1e-5))
    u = W2.T @ g2
    gu = g1 * u
    w_eff = (c * c) * (W1.T @ gu)
    const = ((c * c) * jnp.dot(b1, gu) + c * jnp.dot(be1, u)
             + c * jnp.dot(b2, g2) + jnp.sum(be2) + bias[0])

    out = _deepfm_sc(fm2, fm1r, idx_fm, xv_sm, xv_fm, w_eff, B=B)
    return out + const
